# baseline (speedup 1.0000x reference)
import functools

import jax
import jax.numpy as jnp
from jax.experimental import pallas as pl
from jax.experimental.pallas import tpu as pltpu

_BN_EPS = 1e-5
_NEG_SLOPE = 0.01


def _fused_body(x_ref, w_ref, g_ref, b_ref, o_ref,
                y_scr, wt_scr, sum_scr, ssq_scr, a_scr, bias_scr, *, m):
    p = pl.program_id(0)
    s = pl.program_id(1)
    rows = x_ref.shape[1] * x_ref.shape[2]
    cin = x_ref.shape[3]
    cout = wt_scr.shape[1]

    @pl.when(p == 0)
    def _stats_phase():
        @pl.when(s == 0)
        def _():
            wt_scr[...] = w_ref[...].reshape(cout, cin).T
            sum_scr[...] = jnp.zeros_like(sum_scr)
            ssq_scr[...] = jnp.zeros_like(ssq_scr)

        x2 = x_ref[0].reshape(rows, cin)
        y = jax.lax.dot_general(
            x2, wt_scr[...], (((1,), (0,)), ((), ())),
            preferred_element_type=jnp.float32)
        y_scr[s] = y
        y8 = y.reshape(rows // 8, 8, cout)
        sum_scr[...] += jnp.sum(y8, axis=0)
        ssq_scr[...] += jnp.sum(y8 * y8, axis=0)

    @pl.when(p == 1)
    def _apply_phase():
        @pl.when(s == 0)
        def _():
            inv_m = 1.0 / m
            mean = jnp.sum(sum_scr[...], axis=0, keepdims=True) * inv_m
            ey2 = jnp.sum(ssq_scr[...], axis=0, keepdims=True) * inv_m
            var = jnp.maximum(ey2 - mean * mean, 0.0)
            a = g_ref[...] * jax.lax.rsqrt(var + _BN_EPS)
            a_scr[...] = a
            bias_scr[...] = b_ref[...] - a * mean

        z = y_scr[s] * a_scr[...] + bias_scr[...]
        z = jnp.maximum(z, _NEG_SLOPE * z)
        o_ref[0] = z.reshape(o_ref.shape[1], o_ref.shape[2], o_ref.shape[3])


@jax.jit
def _conv_bn_leaky(x_nchw, conv_w, gamma, beta):
    N, Cin, H, W = x_nchw.shape
    Cout = conv_w.shape[0]
    m = float(N * H * W)
    h_blk = H // 2 if H % 16 == 0 else H
    hb = H // h_blk
    steps = N * hb

    xt = jnp.transpose(x_nchw, (0, 2, 3, 1))
    w_r = conv_w.reshape(Cout * Cin // 128, 128)
    g1 = gamma.astype(jnp.float32).reshape(1, Cout)
    b1 = beta.astype(jnp.float32).reshape(1, Cout)

    body = functools.partial(_fused_body, m=m)
    out_t = pl.pallas_call(
        body,
        out_shape=jax.ShapeDtypeStruct((N, H, W, Cout), x_nchw.dtype),
        grid=(2, steps),
        in_specs=[
            pl.BlockSpec(
                (1, h_blk, W, Cin),
                lambda p, s: ((s // hb) * (1 - p) + (N - 1) * p,
                              (s % hb) * (1 - p) + (hb - 1) * p, 0, 0)),
            pl.BlockSpec((Cout * Cin // 128, 128), lambda p, s: (0, 0)),
            pl.BlockSpec((1, Cout), lambda p, s: (0, 0)),
            pl.BlockSpec((1, Cout), lambda p, s: (0, 0)),
        ],
        out_specs=pl.BlockSpec(
            (1, h_blk, W, Cout),
            lambda p, s: (p * (s // hb), p * (s % hb), 0, 0)),
        scratch_shapes=[
            pltpu.VMEM((steps, h_blk * W, Cout), jnp.float32),
            pltpu.VMEM((Cin, Cout), jnp.float32),
            pltpu.VMEM((8, Cout), jnp.float32),
            pltpu.VMEM((8, Cout), jnp.float32),
            pltpu.VMEM((1, Cout), jnp.float32),
            pltpu.VMEM((1, Cout), jnp.float32),
        ],
        compiler_params=pltpu.CompilerParams(
            dimension_semantics=("arbitrary", "arbitrary"),
            vmem_limit_bytes=58 * 1024 * 1024),
    )(xt, w_r, g1, b1)

    return jnp.transpose(out_t, (0, 3, 1, 2))


def kernel(x_nchw, conv_w, gamma, beta):
    return [_conv_bn_leaky(x_nchw, conv_w, gamma, beta)]


# speedup vs baseline: 1.2751x; 1.2751x over previous
import functools

import jax
import jax.numpy as jnp
from jax.experimental import pallas as pl
from jax.experimental.pallas import tpu as pltpu

_BN_EPS = 1e-5
_NEG_SLOPE = 0.01


def _fused_body(x_ref, w_ref, g_ref, b_ref, o_ref,
                y_scr, wt_scr, sum_scr, ssq_scr, a_scr, bias_scr, *, m):
    p = pl.program_id(0)
    s = pl.program_id(1)
    rows = x_ref.shape[1] * x_ref.shape[2]
    cin = x_ref.shape[3]
    cout = wt_scr.shape[1]

    @pl.when(p == 0)
    def _stats_phase():
        @pl.when(s == 0)
        def _():
            wt_scr[...] = w_ref[...].reshape(cout, cin).T
            sum_scr[...] = jnp.zeros_like(sum_scr)
            ssq_scr[...] = jnp.zeros_like(ssq_scr)

        x2 = x_ref[0].reshape(rows, cin)
        y = jax.lax.dot_general(
            x2, wt_scr[...], (((1,), (0,)), ((), ())),
            preferred_element_type=jnp.float32)
        y_scr[s] = y
        y8 = y.reshape(rows // 8, 8, cout)
        sum_scr[...] += jnp.sum(y8, axis=0)
        ssq_scr[...] += jnp.sum(y8 * y8, axis=0)

    @pl.when(p == 1)
    def _apply_phase():
        @pl.when(s == 0)
        def _():
            inv_m = 1.0 / m
            mean = jnp.sum(sum_scr[...], axis=0, keepdims=True) * inv_m
            ey2 = jnp.sum(ssq_scr[...], axis=0, keepdims=True) * inv_m
            var = jnp.maximum(ey2 - mean * mean, 0.0)
            a = g_ref[...] * jax.lax.rsqrt(var + _BN_EPS)
            a_scr[...] = a
            bias_scr[...] = b_ref[...] - a * mean

        z = y_scr[s] * a_scr[...] + bias_scr[...]
        z = jnp.maximum(z, _NEG_SLOPE * z)
        o_ref[0] = z.reshape(o_ref.shape[1], o_ref.shape[2], o_ref.shape[3])


@jax.jit
def _conv_bn_leaky(x_nchw, conv_w, gamma, beta):
    N, Cin, H, W = x_nchw.shape
    Cout = conv_w.shape[0]
    m = float(N * H * W)
    h_blk = H
    hb = H // h_blk
    steps = N * hb

    xt = jnp.transpose(x_nchw, (0, 2, 3, 1))
    w_r = conv_w.reshape(Cout * Cin // 128, 128)
    g1 = gamma.astype(jnp.float32).reshape(1, Cout)
    b1 = beta.astype(jnp.float32).reshape(1, Cout)

    body = functools.partial(_fused_body, m=m)
    out_t = pl.pallas_call(
        body,
        out_shape=jax.ShapeDtypeStruct((N, H, W, Cout), x_nchw.dtype),
        grid=(2, steps),
        in_specs=[
            pl.BlockSpec(
                (1, h_blk, W, Cin),
                lambda p, s: ((s // hb) * (1 - p) + (N - 1) * p,
                              (s % hb) * (1 - p) + (hb - 1) * p, 0, 0)),
            pl.BlockSpec((Cout * Cin // 128, 128), lambda p, s: (0, 0)),
            pl.BlockSpec((1, Cout), lambda p, s: (0, 0)),
            pl.BlockSpec((1, Cout), lambda p, s: (0, 0)),
        ],
        out_specs=pl.BlockSpec(
            (1, h_blk, W, Cout),
            lambda p, s: (p * (s // hb), p * (s % hb), 0, 0)),
        scratch_shapes=[
            pltpu.VMEM((steps, h_blk * W, Cout), jnp.float32),
            pltpu.VMEM((Cin, Cout), jnp.float32),
            pltpu.VMEM((8, Cout), jnp.float32),
            pltpu.VMEM((8, Cout), jnp.float32),
            pltpu.VMEM((1, Cout), jnp.float32),
            pltpu.VMEM((1, Cout), jnp.float32),
        ],
        compiler_params=pltpu.CompilerParams(
            dimension_semantics=("arbitrary", "arbitrary"),
            vmem_limit_bytes=58 * 1024 * 1024),
    )(xt, w_r, g1, b1)

    return jnp.transpose(out_t, (0, 3, 1, 2))


def kernel(x_nchw, conv_w, gamma, beta):
    return [_conv_bn_leaky(x_nchw, conv_w, gamma, beta)]
